# Initial kernel scaffold; baseline (speedup 1.0000x reference)
import jax
import jax.numpy as jnp
from jax.experimental import pallas as pl


def kernel(pred, poses, ranks):
    raise NotImplementedError("write your pallas kernel here")


# speedup vs baseline: 188.7339x; 188.7339x over previous
import functools

import jax
import jax.numpy as jnp
from jax import lax
from jax.experimental import pallas as pl
from jax.experimental.pallas import tpu as pltpu
from jax.experimental.pallas import tpu_sc as plsc

_MAP_H = 64
_MAP_W = 128
_L1_REG = 0.1
_WEIGHT = 1.0


def _make_sc_gather(B, N, T, H, W):
    info = plsc.get_sparse_core_info()
    NC, NS, L = info.num_cores, info.num_subcores, info.num_lanes
    NW = NC * NS
    total_traj = B * N
    traj_per_w = total_traj // NW
    assert total_traj % NW == 0 and traj_per_w % 8 == 0
    n_full = T // L
    tail = T - n_full * L
    mesh = plsc.VectorSubcoreMesh(core_axis_name="c", subcore_axis_name="s")

    @functools.partial(
        pl.kernel,
        mesh=mesh,
        compiler_params=pltpu.CompilerParams(needs_layout_passes=False),
        out_type=jax.ShapeDtypeStruct((total_traj,), jnp.float32),
        scratch_types=[
            pltpu.VMEM((H * W,), jnp.float32),
            pltpu.VMEM((traj_per_w, T), jnp.int32),
            pltpu.VMEM((traj_per_w, T), jnp.int32),
            pltpu.VMEM((traj_per_w,), jnp.float32),
        ],
    )
    def sc_gather(pred_hbm, rows_hbm, cols_hbm, out_hbm,
                  pred_v, rows_v, cols_v, reward_v):
        wid = lax.axis_index("s") * NC + lax.axis_index("c")
        base = wid * traj_per_w
        b = base // N
        pltpu.sync_copy(pred_hbm.at[b], pred_v)
        pltpu.sync_copy(rows_hbm.at[pl.ds(base, traj_per_w)], rows_v)
        pltpu.sync_copy(cols_hbm.at[pl.ds(base, traj_per_w)], cols_v)

        lane = lax.iota(jnp.int32, L)
        tail_mask = lane >= (L - tail)
        write_mask = lane == 0

        def body(n, carry):
            acc = jnp.zeros((L,), jnp.float32)
            for j in range(n_full):
                r = rows_v[n, pl.ds(j * L, L)]
                c = cols_v[n, pl.ds(j * L, L)]
                acc = acc + plsc.load_gather(pred_v, [r * W + c])
            if tail:
                r = rows_v[n, pl.ds(T - L, L)]
                c = cols_v[n, pl.ds(T - L, L)]
                g = plsc.load_gather(pred_v, [r * W + c])
                acc = acc + jnp.where(tail_mask, g, 0.0)
            total = jnp.sum(acc)
            plsc.store_scatter(reward_v, [jnp.full((L,), n, jnp.int32)],
                               jnp.full((L,), total, jnp.float32),
                               mask=write_mask)
            return carry

        lax.fori_loop(0, traj_per_w, body, 0)
        pltpu.sync_copy(reward_v, out_hbm.at[pl.ds(base, traj_per_w)])

    return sc_gather


def _tc_loss_body(pred_ref, reward_ref, ranks_ref, out_ref, *, B, N, n_elem):
    predv = pred_ref[...]
    l1 = jnp.sum(jnp.abs(predv)) / n_elem

    ranks = ranks_ref[...]
    reward = reward_ref[...]
    pref = ranks == 0
    nprf = ranks > 0
    preff = pref.astype(jnp.float32)
    nprff = nprf.astype(jnp.float32)

    ii = lax.broadcasted_iota(jnp.int32, (N, N), 0)
    jj = lax.broadcasted_iota(jnp.int32, (N, N), 1)
    tri = (ii <= jj).astype(jnp.float32)
    pos_p = lax.dot(preff, tri).astype(jnp.int32) - 1
    pos_q = lax.dot(nprff, tri).astype(jnp.int32) - 1

    Np = jnp.sum(preff, axis=1, keepdims=True)
    Nn = jnp.sum(nprff, axis=1, keepdims=True)
    Npi = Np.astype(jnp.int32)
    Nni = Nn.astype(jnp.int32)

    def gcd_step(_, xy):
        x, y = xy
        cont = y > 0
        return (jnp.where(cont, y, x),
                jnp.where(cont, lax.rem(x, jnp.maximum(y, 1)), 0))

    gi, _ = lax.fori_loop(0, 12, gcd_step, (Npi, Nni))
    gsafe = jnp.maximum(gi, 1)
    rp = lax.rem(jnp.maximum(pos_p, 0), gsafe)
    rq = lax.rem(jnp.maximum(pos_q, 0), gsafe)

    A = reward[:, :, None]
    C = reward[:, None, :]
    m2 = jnp.maximum(A, C)
    nc = m2 + jnp.log1p(jnp.exp(-jnp.abs(A - C)))
    ap = A - nc
    cp = C - nc
    p1 = ap / (ap + cp + 1e-6)

    eq = rp.astype(jnp.float32)[:, :, None] == rq.astype(jnp.float32)[:, None, :]
    wf = (preff[:, :, None] * nprff[:, None, :]) * jnp.where(eq, 1.0, 0.0)
    w = wf > 0
    gf = gi.astype(jnp.float32)[:, :, None]

    nv = (Np * Nn)[:, 0]
    has = nv > 0
    S = jnp.sum(wf * p1 * gf, axis=(1, 2))
    M = jnp.max(jnp.where(w, p1, -jnp.inf), axis=(1, 2))
    Msafe = jnp.where(has, M, 0.0)
    Z = jnp.sum(wf * jnp.exp(p1 - Msafe[:, None, None]), axis=(1, 2)) * gf[:, 0, 0]
    logZ = jnp.where(has, jnp.log(jnp.maximum(Z, 1e-30)), 0.0)
    cls = jnp.where(has, nv * (Msafe + logZ) - S, 0.0)

    total = jnp.sum(cls)
    pairs = jnp.sum(nv)
    out_ref[0, 0] = _WEIGHT * total / (pairs + _L1_REG * l1)


def kernel(pred, poses, ranks):
    B, _, H, W = pred.shape
    N = poses.shape[1]
    T = poses.shape[2]
    pred3 = pred.reshape(B, H * W)
    rows = poses[..., 0].reshape(B * N, T)
    cols = poses[..., 1].reshape(B * N, T)

    reward = _make_sc_gather(B, N, T, H, W)(pred3, rows, cols).reshape(B, N)

    out = pl.pallas_call(
        functools.partial(_tc_loss_body, B=B, N=N, n_elem=float(B * H * W)),
        out_shape=jax.ShapeDtypeStruct((1, 1), jnp.float32),
        out_specs=pl.BlockSpec(memory_space=pltpu.SMEM),
    )(pred.reshape(B * H, W), reward, ranks)
    return out.reshape(())
